# baseline (speedup 1.0000x reference)
import functools

import jax
import jax.numpy as jnp
from jax import lax
from jax.experimental import pallas as pl
from jax.experimental.pallas import tpu as pltpu
from jax.experimental.pallas import tpu_sc as plsc

_NC, _NS, _L = 2, 16, 16
_NW = _NC * _NS
_V = 30522
_VPAD = 30528
_NEG = -3.0e38


def _sc_body(qflat_hbm, qid_hbm, qw_hbm, d_hbm, didf_hbm, dwf_hbm,
             qcls_hbm, dcls_hbm, ztab_hbm, out_hbm,
             table, didb, dwb, qbuf, qidb, qwb, clsq, clsd,
             hits, rows, mbuf, cbuf, outb, sem,
             *, n_per_w, ld, kdld):
    iot = lax.iota(jnp.int32, _L)
    wid = lax.axis_index("c") * _NS + lax.axis_index("s")

    b0 = wid * n_per_w
    cps = [pltpu.async_copy(ztab_hbm, table, sem),
           pltpu.async_copy(didf_hbm.at[pl.ds(b0, n_per_w)], didb, sem),
           pltpu.async_copy(dwf_hbm.at[pl.ds(b0, n_per_w)], dwb, sem),
           pltpu.async_copy(qid_hbm.at[pl.ds(b0, n_per_w)], qidb, sem),
           pltpu.async_copy(qw_hbm.at[pl.ds(b0, n_per_w)], qwb, sem),
           pltpu.async_copy(qflat_hbm.at[pl.ds(b0, n_per_w)], qbuf, sem),
           pltpu.async_copy(qcls_hbm.at[pl.ds(b0, n_per_w)], clsq, sem),
           pltpu.async_copy(dcls_hbm.at[pl.ds(b0, n_per_w)], clsd, sem)]
    for cp in cps:
        cp.wait()

    outv = jnp.zeros((_L,), jnp.float32)
    for t in range(n_per_w):
        b = b0 + t

        qv0 = qidb[t, pl.ds(0, _L)]
        qv1 = qidb[t, pl.ds(_L, _L)]
        one = jnp.ones((_L,), jnp.int32)
        plsc.store_scatter(table, [qv0], one)
        plsc.store_scatter(table, [qv1], one)

        cacc = jnp.zeros((_L,), jnp.float32)
        for k in range(128 // _L):
            cacc = cacc + (clsq[t, pl.ds(k * _L, _L)] *
                           clsd[t, pl.ds(k * _L, _L)])
        cls_b = jnp.sum(cacc)

        def _probe(cc, count):
            hms = []
            for u in range(4):
                dv = didb[t, pl.ds((cc * 4 + u) * _L, _L)]
                tv = plsc.load_gather(table, [dv])
                hms.append(tv == 1)
            anym = (hms[0] | hms[1]) | (hms[2] | hms[3])

            def _grp(cnt):
                for u in range(4):
                    def _rec(st):
                        hm2, c2 = st
                        l = plsc.all_reduce_ffs(hm2)
                        l = jnp.max(l) if l.ndim else l
                        p = (cc * 4 + u) * _L + l
                        plsc.store_scatter(
                            hits, [jnp.full((_L,), c2, jnp.int32)],
                            jnp.full((_L,), p, jnp.int32), mask=iot == 0)
                        return hm2 & (iot != l), c2 + 1
                    _, cnt = lax.while_loop(lambda s: jnp.any(s[0]), _rec,
                                            (hms[u], cnt))
                return cnt

            return lax.cond(jnp.any(anym), _grp, lambda c2: c2, count)
        count = lax.fori_loop(0, kdld // (4 * _L), _probe, jnp.int32(0))

        for half in range(2):
            mbuf[pl.ds(half * _L, _L)] = jnp.full((_L,), _NEG, jnp.float32)
            cbuf[pl.ds(half * _L, _L)] = jnp.zeros((_L,), jnp.int32)

        def _hitstep(h):
            hb = plsc.load_gather(hits, [jnp.full((_L,), h, jnp.int32)])
            jb = jnp.bitwise_and(hb, ld - 1)
            pltpu.async_copy(d_hbm.at[b].at[jb], rows, sem).wait()
            tfull = jnp.full((_L,), t, jnp.int32)
            xb = plsc.load_gather(didb, [tfull, hb])
            wb = plsc.load_gather(dwb, [tfull, hb])
            for half, qv in ((0, qv0), (1, qv1)):
                qm = qv == xb

                def _imatch(s2):
                    qm2 = s2
                    i16 = plsc.all_reduce_ffs(qm2)
                    i16 = jnp.max(i16) if i16.ndim else i16
                    base = (half * _L + i16) * 128
                    acc = jnp.zeros((_L,), jnp.float32)
                    for k in range(128 // _L):
                        acc = acc + (qbuf[t, pl.ds(base + k * _L, _L)] *
                                     rows[0, pl.ds(k * _L, _L)])
                    val = jnp.sum(acc) * wb
                    oh = iot == i16
                    mold = mbuf[pl.ds(half * _L, _L)]
                    mbuf[pl.ds(half * _L, _L)] = jnp.where(
                        oh, jnp.maximum(mold, val), mold)
                    cold = cbuf[pl.ds(half * _L, _L)]
                    cbuf[pl.ds(half * _L, _L)] = jnp.where(
                        oh, cold + 1, cold)
                    return qm2 & (iot != i16)

                lax.while_loop(lambda s: jnp.any(s), _imatch, qm)
            return h + 1
        lax.while_loop(lambda h: h < count, _hitstep, jnp.int32(0))

        tok = jnp.float32(0.0)
        for half in range(2):
            mh = mbuf[pl.ds(half * _L, _L)]
            ch = cbuf[pl.ds(half * _L, _L)]
            mh = jnp.where(ch < kdld, jnp.maximum(mh, 0.0), mh)
            tok = tok + jnp.sum(qwb[t, pl.ds(half * _L, _L)] * mh)
        outv = jnp.where(iot == t, tok + cls_b, outv)

        zero = jnp.zeros((_L,), jnp.int32)
        plsc.store_scatter(table, [qv0], zero)
        plsc.store_scatter(table, [qv1], zero)

    outb[...] = outv
    pltpu.sync_copy(outb, out_hbm.at[wid])


def kernel(query_expert_repr, query_expert_weights, query_expert_ids,
           doc_expert_repr, doc_expert_weights, doc_expert_ids,
           query_cls_repr, doc_cls_repr):
    B, Lq, D = query_expert_repr.shape
    _, Ld, Kd = doc_expert_ids.shape
    n_per_w = B // _NW
    kdld = Kd * Ld

    qflat = query_expert_repr.reshape(B, Lq * D)
    qid = query_expert_ids.reshape(B, Lq).astype(jnp.int32)
    qw = query_expert_weights.reshape(B, Lq)
    didf = jnp.swapaxes(doc_expert_ids, 1, 2).reshape(B, kdld)
    dwf = jnp.swapaxes(doc_expert_weights, 1, 2).reshape(B, kdld)

    mesh = plsc.VectorSubcoreMesh(core_axis_name="c", subcore_axis_name="s",
                                  num_cores=_NC, num_subcores=_NS)
    body = functools.partial(_sc_body, n_per_w=n_per_w, ld=Ld, kdld=kdld)
    out2 = pl.kernel(
        body,
        out_type=jax.ShapeDtypeStruct((_NW, _L), jnp.float32),
        mesh=mesh,
        compiler_params=pltpu.CompilerParams(needs_layout_passes=False),
        scratch_types=[
            pltpu.VMEM((_VPAD,), jnp.int32),
            pltpu.VMEM((n_per_w, kdld), jnp.int32),
            pltpu.VMEM((n_per_w, kdld), jnp.float32),
            pltpu.VMEM((n_per_w, Lq * D), jnp.float32),
            pltpu.VMEM((n_per_w, Lq), jnp.int32),
            pltpu.VMEM((n_per_w, Lq), jnp.float32),
            pltpu.VMEM((n_per_w, D), jnp.float32),
            pltpu.VMEM((n_per_w, D), jnp.float32),
            pltpu.VMEM((kdld,), jnp.int32),
            pltpu.VMEM((_L, D), jnp.float32),
            pltpu.VMEM((2 * _L,), jnp.float32),
            pltpu.VMEM((2 * _L,), jnp.int32),
            pltpu.VMEM((_L,), jnp.float32),
            pltpu.SemaphoreType.DMA,
        ],
    )(qflat, qid, qw, doc_expert_repr, didf, dwf,
      query_cls_repr, doc_cls_repr, jnp.zeros((_VPAD,), jnp.int32))
    return out2[:, :n_per_w].reshape(B)


# speedup vs baseline: 1.0967x; 1.0967x over previous
import functools

import jax
import jax.numpy as jnp
from jax import lax
from jax.experimental import pallas as pl
from jax.experimental.pallas import tpu as pltpu
from jax.experimental.pallas import tpu_sc as plsc

_NC, _NS, _L = 2, 16, 16
_NW = _NC * _NS
_V = 30522
_VPAD = 30528
_NEG = -3.0e38


def _sc_body(qflat_hbm, qid_hbm, qw_hbm, d_hbm, didf_hbm, dwf_hbm,
             qcls_hbm, dcls_hbm, out_hbm,
             table, didb, dwb, qbuf, qidb, qwb, clsq, clsd,
             hits, rows, mbuf, cbuf, outb, sem,
             *, n_per_w, ld, kdld):
    iot = lax.iota(jnp.int32, _L)
    wid = lax.axis_index("c") * _NS + lax.axis_index("s")

    b0 = wid * n_per_w
    cps = [pltpu.async_copy(didf_hbm.at[pl.ds(b0, n_per_w)], didb, sem),
           pltpu.async_copy(dwf_hbm.at[pl.ds(b0, n_per_w)], dwb, sem),
           pltpu.async_copy(qid_hbm.at[pl.ds(b0, n_per_w)], qidb, sem),
           pltpu.async_copy(qw_hbm.at[pl.ds(b0, n_per_w)], qwb, sem),
           pltpu.async_copy(qflat_hbm.at[pl.ds(b0, n_per_w)], qbuf, sem),
           pltpu.async_copy(qcls_hbm.at[pl.ds(b0, n_per_w)], clsq, sem),
           pltpu.async_copy(dcls_hbm.at[pl.ds(b0, n_per_w)], clsd, sem)]

    def _init(k, c):
        z = jnp.zeros((_L,), jnp.int32)
        for u in range(12):
            table[pl.ds(k * 12 * _L + u * _L, _L)] = z
        return c
    lax.fori_loop(0, _VPAD // (12 * _L), _init, 0)

    for cp in cps:
        cp.wait()

    outv = jnp.zeros((_L,), jnp.float32)
    for t in range(n_per_w):
        b = b0 + t

        qv0 = qidb[t, pl.ds(0, _L)]
        qv1 = qidb[t, pl.ds(_L, _L)]
        one = jnp.ones((_L,), jnp.int32)
        plsc.store_scatter(table, [qv0], one)
        plsc.store_scatter(table, [qv1], one)

        cacc = jnp.zeros((_L,), jnp.float32)
        for k in range(128 // _L):
            cacc = cacc + (clsq[t, pl.ds(k * _L, _L)] *
                           clsd[t, pl.ds(k * _L, _L)])
        cls_b = jnp.sum(cacc)

        def _probe(cc, count):
            hms = []
            for u in range(4):
                dv = didb[t, pl.ds((cc * 4 + u) * _L, _L)]
                tv = plsc.load_gather(table, [dv])
                hms.append(tv == 1)
            anym = (hms[0] | hms[1]) | (hms[2] | hms[3])

            def _grp(cnt):
                for u in range(4):
                    def _rec(st):
                        hm2, c2 = st
                        l = plsc.all_reduce_ffs(hm2)
                        l = jnp.max(l) if l.ndim else l
                        p = (cc * 4 + u) * _L + l
                        plsc.store_scatter(
                            hits, [jnp.full((_L,), c2, jnp.int32)],
                            jnp.full((_L,), p, jnp.int32), mask=iot == 0)
                        return hm2 & (iot != l), c2 + 1
                    _, cnt = lax.while_loop(lambda s: jnp.any(s[0]), _rec,
                                            (hms[u], cnt))
                return cnt

            return lax.cond(jnp.any(anym), _grp, lambda c2: c2, count)
        count = lax.fori_loop(0, kdld // (4 * _L), _probe, jnp.int32(0))

        for half in range(2):
            mbuf[pl.ds(half * _L, _L)] = jnp.full((_L,), _NEG, jnp.float32)
            cbuf[pl.ds(half * _L, _L)] = jnp.zeros((_L,), jnp.int32)

        def _hitstep(h):
            hb = plsc.load_gather(hits, [jnp.full((_L,), h, jnp.int32)])
            jb = jnp.bitwise_and(hb, ld - 1)
            cp = pltpu.async_copy(d_hbm.at[b].at[jb], rows, sem)
            tfull = jnp.full((_L,), t, jnp.int32)
            xb = plsc.load_gather(didb, [tfull, hb])
            wb = plsc.load_gather(dwb, [tfull, hb])
            cp.wait()
            for half, qv in ((0, qv0), (1, qv1)):
                qm = qv == xb

                def _imatch(s2):
                    qm2 = s2
                    i16 = plsc.all_reduce_ffs(qm2)
                    i16 = jnp.max(i16) if i16.ndim else i16
                    base = (half * _L + i16) * 128
                    acc = jnp.zeros((_L,), jnp.float32)
                    for k in range(128 // _L):
                        acc = acc + (qbuf[t, pl.ds(base + k * _L, _L)] *
                                     rows[0, pl.ds(k * _L, _L)])
                    val = jnp.sum(acc) * wb
                    oh = iot == i16
                    mold = mbuf[pl.ds(half * _L, _L)]
                    mbuf[pl.ds(half * _L, _L)] = jnp.where(
                        oh, jnp.maximum(mold, val), mold)
                    cold = cbuf[pl.ds(half * _L, _L)]
                    cbuf[pl.ds(half * _L, _L)] = jnp.where(
                        oh, cold + 1, cold)
                    return qm2 & (iot != i16)

                lax.while_loop(lambda s: jnp.any(s), _imatch, qm)
            return h + 1
        lax.while_loop(lambda h: h < count, _hitstep, jnp.int32(0))

        tok = jnp.float32(0.0)
        for half in range(2):
            mh = mbuf[pl.ds(half * _L, _L)]
            ch = cbuf[pl.ds(half * _L, _L)]
            mh = jnp.where(ch < kdld, jnp.maximum(mh, 0.0), mh)
            tok = tok + jnp.sum(qwb[t, pl.ds(half * _L, _L)] * mh)
        outv = jnp.where(iot == t, tok + cls_b, outv)

        zero = jnp.zeros((_L,), jnp.int32)
        plsc.store_scatter(table, [qv0], zero)
        plsc.store_scatter(table, [qv1], zero)

    outb[...] = outv
    pltpu.sync_copy(outb, out_hbm.at[wid])


def kernel(query_expert_repr, query_expert_weights, query_expert_ids,
           doc_expert_repr, doc_expert_weights, doc_expert_ids,
           query_cls_repr, doc_cls_repr):
    B, Lq, D = query_expert_repr.shape
    _, Ld, Kd = doc_expert_ids.shape
    n_per_w = B // _NW
    kdld = Kd * Ld

    qflat = query_expert_repr.reshape(B, Lq * D)
    qid = query_expert_ids.reshape(B, Lq).astype(jnp.int32)
    qw = query_expert_weights.reshape(B, Lq)
    didf = jnp.swapaxes(doc_expert_ids, 1, 2).reshape(B, kdld)
    dwf = jnp.swapaxes(doc_expert_weights, 1, 2).reshape(B, kdld)

    mesh = plsc.VectorSubcoreMesh(core_axis_name="c", subcore_axis_name="s",
                                  num_cores=_NC, num_subcores=_NS)
    body = functools.partial(_sc_body, n_per_w=n_per_w, ld=Ld, kdld=kdld)
    out2 = pl.kernel(
        body,
        out_type=jax.ShapeDtypeStruct((_NW, _L), jnp.float32),
        mesh=mesh,
        compiler_params=pltpu.CompilerParams(needs_layout_passes=False),
        scratch_types=[
            pltpu.VMEM((_VPAD,), jnp.int32),
            pltpu.VMEM((n_per_w, kdld), jnp.int32),
            pltpu.VMEM((n_per_w, kdld), jnp.float32),
            pltpu.VMEM((n_per_w, Lq * D), jnp.float32),
            pltpu.VMEM((n_per_w, Lq), jnp.int32),
            pltpu.VMEM((n_per_w, Lq), jnp.float32),
            pltpu.VMEM((n_per_w, D), jnp.float32),
            pltpu.VMEM((n_per_w, D), jnp.float32),
            pltpu.VMEM((kdld,), jnp.int32),
            pltpu.VMEM((_L, D), jnp.float32),
            pltpu.VMEM((2 * _L,), jnp.float32),
            pltpu.VMEM((2 * _L,), jnp.int32),
            pltpu.VMEM((_L,), jnp.float32),
            pltpu.SemaphoreType.DMA,
        ],
    )(qflat, qid, qw, doc_expert_repr, didf, dwf,
      query_cls_repr, doc_cls_repr)
    return out2[:, :n_per_w].reshape(B)
